# baseline (speedup 1.0000x reference)
import jax
import jax.numpy as jnp
from jax import lax
from jax.experimental import pallas as pl

EPS = 1e-11

ROWS = 128
COLS = 100000
PACK = 4
SUB = (PACK * COLS) // 128
NEG_INF = -3.0e38


def _gumbel_softmax_kernel(x_ref, u_ref, o_ref):
    x = x_ref[0]
    u = u_ref[0]
    g = -jnp.log(-jnp.log(u + EPS))
    y = x + g

    flat = (
        lax.broadcasted_iota(jnp.int32, (SUB, 128), 0) * 128
        + lax.broadcasted_iota(jnp.int32, (SUB, 128), 1)
    )
    b = [r * COLS for r in range(1, PACK)]
    masks = []
    lo = None
    for r in range(PACK):
        if r == 0:
            m = flat < b[0]
        elif r == PACK - 1:
            m = flat >= b[-1]
        else:
            m = (flat >= b[r - 1]) & (flat < b[r])
        masks.append(m)

    mx = [jnp.max(jnp.where(masks[r], y, NEG_INF)) for r in range(PACK)]
    m_sel = jnp.where(
        flat < b[0], mx[0],
        jnp.where(flat < b[1], mx[1], jnp.where(flat < b[2], mx[2], mx[3])),
    )
    e = jnp.exp(y - m_sel)
    s = [jnp.sum(jnp.where(masks[r], e, 0.0)) for r in range(PACK)]
    inv = [1.0 / v for v in s]
    inv_sel = jnp.where(
        flat < b[0], inv[0],
        jnp.where(flat < b[1], inv[1], jnp.where(flat < b[2], inv[2], inv[3])),
    )
    o_ref[0] = e * inv_sel


def kernel(logits, u):
    xv = logits.reshape(ROWS // PACK, SUB, 128)
    uv = u.reshape(ROWS // PACK, SUB, 128)
    grid = (ROWS // PACK,)
    spec = pl.BlockSpec((1, SUB, 128), lambda i: (i, 0, 0))
    out = pl.pallas_call(
        _gumbel_softmax_kernel,
        grid=grid,
        in_specs=[spec, spec],
        out_specs=spec,
        out_shape=jax.ShapeDtypeStruct((ROWS // PACK, SUB, 128), jnp.float32),
    )(xv, uv)
    return out.reshape(ROWS, COLS)


# speedup vs baseline: 1.2799x; 1.2799x over previous
import jax
import jax.numpy as jnp
from jax import lax
from jax.experimental import pallas as pl
from jax.experimental.pallas import tpu as pltpu

EPS = 1e-11

ROWS = 128
COLS = 100000
BC = 1024
NC = (COLS + BC - 1) // BC
NEG_INF = -3.0e38


def _gs_kernel(x_ref, u_ref, o_ref, y_scr, m_scr, s_scr):
    p = pl.program_id(0)
    c = pl.program_id(1)

    @pl.when(p == 0)
    def _p0():
        x = x_ref[...]
        u = u_ref[...]
        y = x - jnp.log(-jnp.log(u + EPS))
        gcol = c * BC + lax.broadcasted_iota(jnp.int32, (ROWS, BC), 1)
        y = jnp.where(gcol < COLS, y, NEG_INF)
        bm = jnp.max(y, axis=1, keepdims=True)

        @pl.when(c == 0)
        def _():
            m_scr[...] = bm

        @pl.when(c > 0)
        def _():
            m_scr[...] = jnp.maximum(m_scr[...], bm)

        y_scr[:, pl.ds(c * BC, BC)] = y

    @pl.when(p == 1)
    def _p1():
        y = y_scr[:, pl.ds(c * BC, BC)]
        e = jnp.exp(y - m_scr[...])
        bs = jnp.sum(e, axis=1, keepdims=True)

        @pl.when(c == 0)
        def _():
            s_scr[...] = bs

        @pl.when(c > 0)
        def _():
            s_scr[...] = s_scr[...] + bs

        y_scr[:, pl.ds(c * BC, BC)] = e

    @pl.when(p == 2)
    def _p2():
        e = y_scr[:, pl.ds(c * BC, BC)]
        o_ref[...] = e * (1.0 / s_scr[...])


def kernel(logits, u):
    in_spec = pl.BlockSpec(
        (ROWS, BC), lambda p, c: (0, jnp.where(p == 0, c, 0))
    )
    out_spec = pl.BlockSpec(
        (ROWS, BC), lambda p, c: (0, jnp.where(p == 2, c, 0))
    )
    return pl.pallas_call(
        _gs_kernel,
        grid=(3, NC),
        in_specs=[in_spec, in_spec],
        out_specs=out_spec,
        out_shape=jax.ShapeDtypeStruct((ROWS, COLS), jnp.float32),
        scratch_shapes=[
            pltpu.VMEM((ROWS, NC * BC), jnp.float32),
            pltpu.VMEM((ROWS, 1), jnp.float32),
            pltpu.VMEM((ROWS, 1), jnp.float32),
        ],
    )(logits, u)


# speedup vs baseline: 1.6101x; 1.2580x over previous
import jax
import jax.numpy as jnp
from jax import lax
from jax.experimental import pallas as pl
from jax.experimental.pallas import tpu as pltpu

EPS = 1e-11

ROWS = 128
COLS = 100000
BC = 2048
NC = (COLS + BC - 1) // BC
NEG_INF = -3.0e38


def _gs_kernel(x_ref, u_ref, o_ref, y_scr, m_scr, s_scr):
    p = pl.program_id(0)
    c = pl.program_id(1)

    @pl.when(p == 0)
    def _p0():
        x = x_ref[...]
        u = u_ref[...]
        y = x - jnp.log(-jnp.log(u + EPS))
        gcol = c * BC + lax.broadcasted_iota(jnp.int32, (ROWS, BC), 1)
        y = jnp.where(gcol < COLS, y, NEG_INF)
        bm = jnp.max(y, axis=1, keepdims=True)

        @pl.when(c == 0)
        def _():
            m_scr[...] = bm
            s_scr[...] = jnp.sum(jnp.exp(y - bm), axis=1, keepdims=True)

        @pl.when(c > 0)
        def _():
            m_old = m_scr[...]
            m_new = jnp.maximum(m_old, bm)
            bs = jnp.sum(jnp.exp(y - m_new), axis=1, keepdims=True)
            s_scr[...] = s_scr[...] * jnp.exp(m_old - m_new) + bs
            m_scr[...] = m_new

        y_scr[:, pl.ds(c * BC, BC)] = y

    @pl.when(p == 1)
    def _p1():
        y = y_scr[:, pl.ds(c * BC, BC)]
        e = jnp.exp(y - m_scr[...])
        o_ref[...] = e * (1.0 / s_scr[...])


def kernel(logits, u):
    in_spec = pl.BlockSpec(
        (ROWS, BC), lambda p, c: (0, jnp.where(p == 0, c, 0))
    )
    out_spec = pl.BlockSpec(
        (ROWS, BC), lambda p, c: (0, jnp.where(p == 1, c, 0))
    )
    return pl.pallas_call(
        _gs_kernel,
        grid=(2, NC),
        in_specs=[in_spec, in_spec],
        out_specs=out_spec,
        out_shape=jax.ShapeDtypeStruct((ROWS, COLS), jnp.float32),
        scratch_shapes=[
            pltpu.VMEM((ROWS, NC * BC), jnp.float32),
            pltpu.VMEM((ROWS, 1), jnp.float32),
            pltpu.VMEM((ROWS, 1), jnp.float32),
        ],
    )(logits, u)


# speedup vs baseline: 4.1414x; 2.5721x over previous
import jax
import jax.numpy as jnp
from jax.experimental import pallas as pl
from jax.experimental.pallas import tpu as pltpu

EPS = 1e-11

ROWS = 128
COLS = 100000
BS = 2000
NB = COLS // BS


def _gs_kernel(x_ref, u_ref, o_ref, y_scr, m_scr, s_scr):
    p = pl.program_id(0)
    c = pl.program_id(1)

    @pl.when(p == 0)
    def _p0():
        y = x_ref[...] - jnp.log(-jnp.log(u_ref[...] + EPS))
        bm = jnp.max(y, axis=0, keepdims=True)

        @pl.when(c == 0)
        def _():
            m_scr[...] = bm
            s_scr[...] = jnp.sum(jnp.exp(y - bm), axis=0, keepdims=True)

        @pl.when(c > 0)
        def _():
            m_old = m_scr[...]
            m_new = jnp.maximum(m_old, bm)
            bs = jnp.sum(jnp.exp(y - m_new), axis=0, keepdims=True)
            s_scr[...] = s_scr[...] * jnp.exp(m_old - m_new) + bs
            m_scr[...] = m_new

        y_scr[pl.ds(c * BS, BS), :] = y

    @pl.when(p == 1)
    def _p1():
        e = jnp.exp(y_scr[pl.ds(c * BS, BS), :] - m_scr[...])
        o_ref[...] = e * (1.0 / s_scr[...])


def kernel(logits, u):
    xt = logits.T
    ut = u.T
    in_spec = pl.BlockSpec(
        (BS, ROWS), lambda p, c: (jnp.where(p == 0, c, 0), 0)
    )
    out_spec = pl.BlockSpec(
        (BS, ROWS), lambda p, c: (jnp.where(p == 1, c, 0), 0)
    )
    out = pl.pallas_call(
        _gs_kernel,
        grid=(2, NB),
        in_specs=[in_spec, in_spec],
        out_specs=out_spec,
        out_shape=jax.ShapeDtypeStruct((COLS, ROWS), jnp.float32),
        scratch_shapes=[
            pltpu.VMEM((COLS, ROWS), jnp.float32),
            pltpu.VMEM((1, ROWS), jnp.float32),
            pltpu.VMEM((1, ROWS), jnp.float32),
        ],
    )(xt, ut)
    return out.T


# speedup vs baseline: 4.1658x; 1.0059x over previous
import jax
import jax.numpy as jnp
from jax.experimental import pallas as pl
from jax.experimental.pallas import tpu as pltpu

EPS = 1e-11

ROWS = 128
COLS = 100000
BS = 1000
NP = COLS // (2 * BS)


def _gs_kernel(xe_ref, xo_ref, ue_ref, uo_ref, o_ref, y_scr, m_scr, s_scr):
    p = pl.program_id(0)
    c = pl.program_id(1)

    @pl.when(p == 0)
    def _p0():
        ye = xe_ref[...] - jnp.log(-jnp.log(ue_ref[...] + EPS))
        yo = xo_ref[...] - jnp.log(-jnp.log(uo_ref[...] + EPS))
        bm = jnp.maximum(
            jnp.max(ye, axis=0, keepdims=True),
            jnp.max(yo, axis=0, keepdims=True),
        )

        @pl.when(c == 0)
        def _():
            m_scr[...] = bm
            s_scr[...] = (
                jnp.sum(jnp.exp(ye - bm), axis=0, keepdims=True)
                + jnp.sum(jnp.exp(yo - bm), axis=0, keepdims=True)
            )

        @pl.when(c > 0)
        def _():
            m_old = m_scr[...]
            m_new = jnp.maximum(m_old, bm)
            bs = (
                jnp.sum(jnp.exp(ye - m_new), axis=0, keepdims=True)
                + jnp.sum(jnp.exp(yo - m_new), axis=0, keepdims=True)
            )
            s_scr[...] = s_scr[...] * jnp.exp(m_old - m_new) + bs
            m_scr[...] = m_new

        y_scr[pl.ds((2 * c) * BS, BS), :] = ye
        y_scr[pl.ds((2 * c + 1) * BS, BS), :] = yo

    @pl.when(p == 1)
    def _p1():
        e = jnp.exp(y_scr[pl.ds(c * 2 * BS, 2 * BS), :] - m_scr[...])
        o_ref[...] = e * (1.0 / s_scr[...])


def kernel(logits, u):
    xt = logits.T
    ut = u.T
    even_in = pl.BlockSpec(
        (BS, ROWS), lambda p, c: (jnp.where(p == 0, 2 * c, 0), 0)
    )
    odd_in = pl.BlockSpec(
        (BS, ROWS), lambda p, c: (jnp.where(p == 0, 2 * c + 1, 1), 0)
    )
    out_spec = pl.BlockSpec(
        (2 * BS, ROWS), lambda p, c: (jnp.where(p == 1, c, 0), 0)
    )
    out = pl.pallas_call(
        _gs_kernel,
        grid=(2, NP),
        in_specs=[even_in, odd_in, even_in, odd_in],
        out_specs=out_spec,
        out_shape=jax.ShapeDtypeStruct((COLS, ROWS), jnp.float32),
        scratch_shapes=[
            pltpu.VMEM((COLS, ROWS), jnp.float32),
            pltpu.VMEM((1, ROWS), jnp.float32),
            pltpu.VMEM((1, ROWS), jnp.float32),
        ],
    )(xt, xt, ut, ut)
    return out.T


# speedup vs baseline: 7.2802x; 1.7476x over previous
import jax
import jax.numpy as jnp
from jax import lax
from jax.experimental import pallas as pl
from jax.experimental.pallas import tpu as pltpu

EPS = 1e-11

ROWS = 128
COLS = 100000
BS = 1000
NB = COLS // BS
XLOOK = 12
UK = 10
OK_ = 8
NEG_INF = -3.0e38


def _gs_kernel(x_hbm, u_hbm, o_hbm, y_scr, ubuf, obuf, x_sem, u_sem, o_sem):
    def xcopy(c):
        return pltpu.make_async_copy(
            x_hbm.at[pl.ds(c * BS, BS), :],
            y_scr.at[pl.ds(c * BS, BS), :],
            x_sem.at[c],
        )

    def ucopy(c, slot):
        return pltpu.make_async_copy(
            u_hbm.at[pl.ds(c * BS, BS), :], ubuf.at[slot], u_sem.at[slot]
        )

    def ocopy(c, slot):
        return pltpu.make_async_copy(
            obuf.at[slot], o_hbm.at[pl.ds(c * BS, BS), :], o_sem.at[slot]
        )

    for c in range(XLOOK):
        xcopy(c).start()
    for c in range(UK):
        ucopy(c, c).start()

    def body_a(c, carry):
        m, s = carry
        slot = lax.rem(c, UK)
        xcopy(c).wait()
        ucopy(c, slot).wait()
        xb = y_scr[pl.ds(c * BS, BS), :]
        ub = ubuf[slot]
        y = xb - jnp.log(-jnp.log(ub + EPS))
        y_scr[pl.ds(c * BS, BS), :] = y
        bm = jnp.max(y, axis=0, keepdims=True)
        m_new = jnp.maximum(m, bm)
        bs = jnp.sum(jnp.exp(y - m_new), axis=0, keepdims=True)
        s = s * jnp.exp(m - m_new) + bs

        @pl.when(c + XLOOK < NB)
        def _():
            xcopy(c + XLOOK).start()

        @pl.when(c + UK < NB)
        def _():
            ucopy(c + UK, slot).start()

        return m_new, s

    m0 = jnp.full((1, ROWS), NEG_INF, jnp.float32)
    s0 = jnp.zeros((1, ROWS), jnp.float32)
    m, s = lax.fori_loop(0, NB, body_a, (m0, s0))
    inv = 1.0 / s

    def body_b(c, _):
        slot = lax.rem(c, OK_)

        @pl.when(c >= OK_)
        def _():
            ocopy(c - OK_, slot).wait()

        e = jnp.exp(y_scr[pl.ds(c * BS, BS), :] - m) * inv
        obuf[slot] = e
        ocopy(c, slot).start()
        return 0

    lax.fori_loop(0, NB, body_b, 0)
    for c in range(NB - OK_, NB):
        ocopy(c, c % OK_).wait()


def kernel(logits, u):
    xt = logits.T
    ut = u.T
    hbm = pl.BlockSpec(memory_space=pltpu.MemorySpace.HBM)
    out = pl.pallas_call(
        _gs_kernel,
        in_specs=[hbm, hbm],
        out_specs=hbm,
        out_shape=jax.ShapeDtypeStruct((COLS, ROWS), jnp.float32),
        scratch_shapes=[
            pltpu.VMEM((COLS, ROWS), jnp.float32),
            pltpu.VMEM((UK, BS, ROWS), jnp.float32),
            pltpu.VMEM((OK_, BS, ROWS), jnp.float32),
            pltpu.SemaphoreType.DMA((NB,)),
            pltpu.SemaphoreType.DMA((UK,)),
            pltpu.SemaphoreType.DMA((OK_,)),
        ],
    )(xt, ut)
    return out.T
